# baseline (speedup 1.0000x reference)
import jax
import jax.numpy as jnp
from jax import lax
from jax.experimental import pallas as pl
from jax.experimental.pallas import tpu as pltpu
from jax.experimental.pallas import tpu_sc as plsc

N_NODES = 10000
N_EDGES = 320000
NPAD = 10240
EPAD = 327680
DUMMY = NPAD - 1
H = 32

NC, NS = 2, 16
NW = NC * NS
EROWS_PER_TILE = EPAD // NW // 128
CH = 10
NROWS_PER_TILE = NPAD // NS


NCHUNK = EROWS_PER_TILE // CH
NBUF = 2


def _sc_scatter_body(src_hbm, dst_hbm, z_hbm, zero_hbm, out_hbm,
                     idx_s, idx_d, rows, accum, gsem, ssem):
    c = lax.axis_index("c")
    s = lax.axis_index("s")
    tslice = pl.ds(s * NROWS_PER_TILE, NROWS_PER_TILE)

    zcp = pltpu.async_copy(zero_hbm.at[tslice], accum.at[tslice], gsem)
    row0 = (c * NS + s) * EROWS_PER_TILE
    pltpu.sync_copy(src_hbm.at[pl.ds(row0, EROWS_PER_TILE)], idx_s)
    pltpu.sync_copy(dst_hbm.at[pl.ds(row0, EROWS_PER_TILE)], idx_d)
    zcp.wait()
    plsc.subcore_barrier()

    gathers = {}
    adds = {}

    def fire(ci):
        buf = rows.at[ci % NBUF]
        gathers[ci] = [
            pltpu.async_copy(z_hbm.at[idx_s.at[ci * CH + j]], buf.at[j], gsem)
            for j in range(CH)]

    for ci in range(NBUF - 1):
        fire(ci)
    for ci in range(NCHUNK):
        buf = rows.at[ci % NBUF]
        for cp in gathers.pop(ci):
            cp.wait()
        if ci >= 1:
            for cp in adds.pop(ci - 1):
                cp.wait()
        adds[ci] = [
            pltpu.async_copy(buf.at[j], accum.at[idx_d.at[ci * CH + j]],
                             ssem, add=True)
            for j in range(CH)]
        if ci + NBUF - 1 < NCHUNK:
            fire(ci + NBUF - 1)
    for cp in adds.pop(NCHUNK - 1):
        cp.wait()

    plsc.subcore_barrier()
    pltpu.sync_copy(accum.at[tslice], out_hbm.at[c].at[tslice])


_sc_scatter = pl.kernel(
    _sc_scatter_body,
    out_type=jax.ShapeDtypeStruct((NC, NPAD, H), jnp.float32),
    mesh=plsc.VectorSubcoreMesh(core_axis_name="c", subcore_axis_name="s"),
    scratch_types=[
        pltpu.VMEM((EROWS_PER_TILE, 128), jnp.int32),
        pltpu.VMEM((EROWS_PER_TILE, 128), jnp.int32),
        pltpu.VMEM((NBUF, CH, 128, H), jnp.float32),
        pltpu.VMEM_SHARED((NPAD, H), jnp.float32),
        pltpu.SemaphoreType.DMA,
        pltpu.SemaphoreType.DMA,
    ],
    compiler_params=pltpu.CompilerParams(use_tc_tiling_on_sc=False),
    name="gcn_edge_scatter",
)


def _sc_degree_body(dst_hbm, zero_hbm, out_hbm, idx_d, ones, accum, sem):
    c = lax.axis_index("c")
    s = lax.axis_index("s")
    tslice = pl.ds(s * NROWS_PER_TILE, NROWS_PER_TILE)

    one16 = jnp.ones((16,), jnp.float32)
    def oloop(i, _):
        ones[pl.ds(i * 16, 16)] = one16
        return ()
    lax.fori_loop(0, 128 // 16, oloop, ())
    pltpu.sync_copy(zero_hbm.at[tslice], accum.at[tslice])
    plsc.subcore_barrier()

    row0 = (c * NS + s) * EROWS_PER_TILE

    def chunk(i, _):
        base = row0 + i * CH
        pltpu.sync_copy(dst_hbm.at[pl.ds(base, CH)], idx_d)
        for j in range(CH):
            pltpu.sync_copy(ones, accum.at[idx_d.at[j]], add=True)
        return ()
    lax.fori_loop(0, EROWS_PER_TILE // CH, chunk, ())

    plsc.subcore_barrier()
    pltpu.sync_copy(accum.at[tslice], out_hbm.at[c].at[tslice])


_sc_degree = pl.kernel(
    _sc_degree_body,
    out_type=jax.ShapeDtypeStruct((NC, NPAD), jnp.float32),
    mesh=plsc.VectorSubcoreMesh(core_axis_name="c", subcore_axis_name="s"),
    scratch_types=[
        pltpu.VMEM((CH, 128), jnp.int32),
        pltpu.VMEM((128,), jnp.float32),
        pltpu.VMEM_SHARED((NPAD,), jnp.float32),
        pltpu.SemaphoreType.DMA,
    ],
    compiler_params=pltpu.CompilerParams(use_tc_tiling_on_sc=False),
    name="gcn_degree",
)


BLK = 1024


def _tc_mm1_body(x_ref, w1_ref, z_ref):
    z_ref[...] = x_ref[...] @ w1_ref[...]


def _tc_mm1(x, w1):
    return pl.pallas_call(
        _tc_mm1_body,
        grid=(NPAD // BLK,),
        in_specs=[
            pl.BlockSpec((BLK, 128), lambda i: (i, 0)),
            pl.BlockSpec((128, H), lambda i: (0, 0)),
        ],
        out_specs=pl.BlockSpec((BLK, H), lambda i: (i, 0)),
        out_shape=jax.ShapeDtypeStruct((NPAD, H), jnp.float32),
        name="gcn_mm1",
    )(x, w1)


def _tc_head_body(z0_ref, d0_ref, d1_ref, z_ref, dinv_ref):
    deg = d0_ref[...] + d1_ref[...] + 1.0
    dinv = lax.rsqrt(deg)
    dinv_ref[...] = dinv
    z_ref[...] = z0_ref[...] * dinv[:, None]


def _tc_head(z0, deg0, deg1):
    return pl.pallas_call(
        _tc_head_body,
        grid=(NPAD // BLK,),
        in_specs=[
            pl.BlockSpec((BLK, H), lambda i: (i, 0)),
            pl.BlockSpec((BLK,), lambda i: (i,)),
            pl.BlockSpec((BLK,), lambda i: (i,)),
        ],
        out_specs=[
            pl.BlockSpec((BLK, H), lambda i: (i, 0)),
            pl.BlockSpec((BLK,), lambda i: (i,)),
        ],
        out_shape=[
            jax.ShapeDtypeStruct((NPAD, H), jnp.float32),
            jax.ShapeDtypeStruct((NPAD,), jnp.float32),
        ],
        name="gcn_head",
    )(z0, deg0, deg1)


def _tc_mid_body(p0_ref, p1_ref, z_ref, dinv_ref, b_ref, w_ref, out_ref):
    dinv = dinv_ref[...]
    h = dinv[:, None] * (p0_ref[...] + p1_ref[...] + z_ref[...]) + b_ref[...]
    out_ref[...] = (h @ w_ref[...]) * dinv[:, None]


def _tc_mid(p, z, dinv, b, w):
    return pl.pallas_call(
        _tc_mid_body,
        grid=(NPAD // BLK,),
        in_specs=[
            pl.BlockSpec((BLK, H), lambda i: (i, 0)),
            pl.BlockSpec((BLK, H), lambda i: (i, 0)),
            pl.BlockSpec((BLK, H), lambda i: (i, 0)),
            pl.BlockSpec((BLK,), lambda i: (i,)),
            pl.BlockSpec((1, H), lambda i: (0, 0)),
            pl.BlockSpec((H, H), lambda i: (0, 0)),
        ],
        out_specs=pl.BlockSpec((BLK, H), lambda i: (i, 0)),
        out_shape=jax.ShapeDtypeStruct((NPAD, H), jnp.float32),
        name="gcn_mid",
    )(p[0], p[1], z, dinv, b, w)


def _tc_pre5_body(p0_ref, p1_ref, z_ref, dinv_ref, b_ref, out_ref):
    dinv = dinv_ref[...]
    h = dinv[:, None] * (p0_ref[...] + p1_ref[...] + z_ref[...]) + b_ref[...]
    out_ref[...] = h * dinv[:, None]


def _tc_pre5(p, z, dinv, b):
    return pl.pallas_call(
        _tc_pre5_body,
        grid=(NPAD // BLK,),
        in_specs=[
            pl.BlockSpec((BLK, H), lambda i: (i, 0)),
            pl.BlockSpec((BLK, H), lambda i: (i, 0)),
            pl.BlockSpec((BLK, H), lambda i: (i, 0)),
            pl.BlockSpec((BLK,), lambda i: (i,)),
            pl.BlockSpec((1, H), lambda i: (0, 0)),
        ],
        out_specs=pl.BlockSpec((BLK, H), lambda i: (i, 0)),
        out_shape=jax.ShapeDtypeStruct((NPAD, H), jnp.float32),
        name="gcn_pre5",
    )(p[0], p[1], z, dinv, b)


def _tc_tail_body(p0_ref, p1_ref, z_ref, dinv_ref, b_ref, w_ref, out_ref):
    dinv = dinv_ref[...]
    g = dinv[:, None] * (p0_ref[...] + p1_ref[...] + z_ref[...])
    out_ref[...] = g @ w_ref[...] + b_ref[...]


def _tc_tail(p, z, dinv, b5, w5):
    return pl.pallas_call(
        _tc_tail_body,
        grid=(NPAD // BLK,),
        in_specs=[
            pl.BlockSpec((BLK, H), lambda i: (i, 0)),
            pl.BlockSpec((BLK, H), lambda i: (i, 0)),
            pl.BlockSpec((BLK, H), lambda i: (i, 0)),
            pl.BlockSpec((BLK,), lambda i: (i,)),
            pl.BlockSpec((1, 128), lambda i: (0, 0)),
            pl.BlockSpec((H, 128), lambda i: (0, 0)),
        ],
        out_specs=pl.BlockSpec((BLK, 128), lambda i: (i, 0)),
        out_shape=jax.ShapeDtypeStruct((NPAD, 128), jnp.float32),
        name="gcn_tail",
    )(p[0], p[1], z, dinv, b5, w5)


@jax.jit
def kernel(x, edge_index, W1, b1, W2, b2, W3, b3, W4, b4, W5, b5):
    src = edge_index[0]
    dst = edge_index[1]
    pad = N_NODES + (jnp.arange(EPAD - N_EDGES, dtype=jnp.int32)
                     % (NPAD - N_NODES))
    src2d = jnp.concatenate([src, pad]).reshape(EPAD // 128, 128)
    dst2d = jnp.concatenate([dst, pad]).reshape(EPAD // 128, 128)
    xpad = jnp.pad(x, ((0, NPAD - N_NODES), (0, 0)))
    zeros2d = jnp.zeros((NPAD, H), jnp.float32)
    zeros1d = jnp.zeros((NPAD,), jnp.float32)

    z0 = _tc_mm1(xpad, W1)
    deg = _sc_degree(dst2d, zeros1d)
    z, dinv = _tc_head(z0, deg[0], deg[1])

    p = _sc_scatter(src2d, dst2d, z, zeros2d)
    z = _tc_mid(p, z, dinv, b1.reshape(1, H), W2)
    p = _sc_scatter(src2d, dst2d, z, zeros2d)
    z = _tc_mid(p, z, dinv, b2.reshape(1, H), W3)
    p = _sc_scatter(src2d, dst2d, z, zeros2d)
    z = _tc_mid(p, z, dinv, b3.reshape(1, H), W4)
    p = _sc_scatter(src2d, dst2d, z, zeros2d)
    z = _tc_pre5(p, z, dinv, b4.reshape(1, H))
    p = _sc_scatter(src2d, dst2d, z, zeros2d)
    out = _tc_tail(p, z, dinv, b5.reshape(1, 128), W5)
    return out[:N_NODES]


# speedup vs baseline: 1.0364x; 1.0364x over previous
import jax
import jax.numpy as jnp
from jax import lax
from jax.experimental import pallas as pl
from jax.experimental.pallas import tpu as pltpu
from jax.experimental.pallas import tpu_sc as plsc

N_NODES = 10000
N_EDGES = 320000
NPAD = 10240
EPAD = 327680
DUMMY = NPAD - 1
H = 32

NC, NS = 2, 16
NW = NC * NS
EROWS_PER_TILE = EPAD // NW // 128
CH = 4
NROWS_PER_TILE = NPAD // NS


NCHUNK = EROWS_PER_TILE // CH
NBUF = 4


def _sc_scatter_body(src_hbm, dst_hbm, z_hbm, zero_hbm, out_hbm,
                     idx_s, idx_d, rows, accum, gsem, ssem):
    c = lax.axis_index("c")
    s = lax.axis_index("s")
    tslice = pl.ds(s * NROWS_PER_TILE, NROWS_PER_TILE)

    zcp = pltpu.async_copy(zero_hbm.at[tslice], accum.at[tslice], gsem)
    row0 = (c * NS + s) * EROWS_PER_TILE
    pltpu.sync_copy(src_hbm.at[pl.ds(row0, EROWS_PER_TILE)], idx_s)
    pltpu.sync_copy(dst_hbm.at[pl.ds(row0, EROWS_PER_TILE)], idx_d)
    zcp.wait()
    plsc.subcore_barrier()

    gathers = {}
    adds = {}

    def fire(ci):
        buf = rows.at[ci % NBUF]
        gathers[ci] = [
            pltpu.async_copy(z_hbm.at[idx_s.at[ci * CH + j]], buf.at[j], gsem)
            for j in range(CH)]

    for ci in range(NBUF - 1):
        fire(ci)
    for ci in range(NCHUNK):
        buf = rows.at[ci % NBUF]
        for cp in gathers.pop(ci):
            cp.wait()
        if ci >= 1:
            for cp in adds.pop(ci - 1):
                cp.wait()
        adds[ci] = [
            pltpu.async_copy(buf.at[j], accum.at[idx_d.at[ci * CH + j]],
                             ssem, add=True)
            for j in range(CH)]
        if ci + NBUF - 1 < NCHUNK:
            fire(ci + NBUF - 1)
    for cp in adds.pop(NCHUNK - 1):
        cp.wait()

    plsc.subcore_barrier()
    pltpu.sync_copy(accum.at[tslice], out_hbm.at[c].at[tslice])


_sc_scatter = pl.kernel(
    _sc_scatter_body,
    out_type=jax.ShapeDtypeStruct((NC, NPAD, H), jnp.float32),
    mesh=plsc.VectorSubcoreMesh(core_axis_name="c", subcore_axis_name="s"),
    scratch_types=[
        pltpu.VMEM((EROWS_PER_TILE, 128), jnp.int32),
        pltpu.VMEM((EROWS_PER_TILE, 128), jnp.int32),
        pltpu.VMEM((NBUF, CH, 128, H), jnp.float32),
        pltpu.VMEM_SHARED((NPAD, H), jnp.float32),
        pltpu.SemaphoreType.DMA,
        pltpu.SemaphoreType.DMA,
    ],
    compiler_params=pltpu.CompilerParams(use_tc_tiling_on_sc=False),
    name="gcn_edge_scatter",
)


def _sc_degree_body(dst_hbm, zero_hbm, out_hbm, idx_d, ones, accum, sem):
    c = lax.axis_index("c")
    s = lax.axis_index("s")
    tslice = pl.ds(s * NROWS_PER_TILE, NROWS_PER_TILE)

    one16 = jnp.ones((16,), jnp.float32)
    def oloop(i, _):
        ones[pl.ds(i * 16, 16)] = one16
        return ()
    lax.fori_loop(0, 128 // 16, oloop, ())
    pltpu.sync_copy(zero_hbm.at[tslice], accum.at[tslice])
    plsc.subcore_barrier()

    row0 = (c * NS + s) * EROWS_PER_TILE

    def chunk(i, _):
        base = row0 + i * CH
        pltpu.sync_copy(dst_hbm.at[pl.ds(base, CH)], idx_d)
        for j in range(CH):
            pltpu.sync_copy(ones, accum.at[idx_d.at[j]], add=True)
        return ()
    lax.fori_loop(0, EROWS_PER_TILE // CH, chunk, ())

    plsc.subcore_barrier()
    pltpu.sync_copy(accum.at[tslice], out_hbm.at[c].at[tslice])


_sc_degree = pl.kernel(
    _sc_degree_body,
    out_type=jax.ShapeDtypeStruct((NC, NPAD), jnp.float32),
    mesh=plsc.VectorSubcoreMesh(core_axis_name="c", subcore_axis_name="s"),
    scratch_types=[
        pltpu.VMEM((CH, 128), jnp.int32),
        pltpu.VMEM((128,), jnp.float32),
        pltpu.VMEM_SHARED((NPAD,), jnp.float32),
        pltpu.SemaphoreType.DMA,
    ],
    compiler_params=pltpu.CompilerParams(use_tc_tiling_on_sc=False),
    name="gcn_degree",
)


BLK = 1024


def _tc_mm1_body(x_ref, w1_ref, z_ref):
    z_ref[...] = x_ref[...] @ w1_ref[...]


def _tc_mm1(x, w1):
    return pl.pallas_call(
        _tc_mm1_body,
        grid=(NPAD // BLK,),
        in_specs=[
            pl.BlockSpec((BLK, 128), lambda i: (i, 0)),
            pl.BlockSpec((128, H), lambda i: (0, 0)),
        ],
        out_specs=pl.BlockSpec((BLK, H), lambda i: (i, 0)),
        out_shape=jax.ShapeDtypeStruct((NPAD, H), jnp.float32),
        name="gcn_mm1",
    )(x, w1)


def _tc_head_body(z0_ref, d0_ref, d1_ref, z_ref, dinv_ref):
    deg = d0_ref[...] + d1_ref[...] + 1.0
    dinv = lax.rsqrt(deg)
    dinv_ref[...] = dinv
    z_ref[...] = z0_ref[...] * dinv[:, None]


def _tc_head(z0, deg0, deg1):
    return pl.pallas_call(
        _tc_head_body,
        grid=(NPAD // BLK,),
        in_specs=[
            pl.BlockSpec((BLK, H), lambda i: (i, 0)),
            pl.BlockSpec((BLK,), lambda i: (i,)),
            pl.BlockSpec((BLK,), lambda i: (i,)),
        ],
        out_specs=[
            pl.BlockSpec((BLK, H), lambda i: (i, 0)),
            pl.BlockSpec((BLK,), lambda i: (i,)),
        ],
        out_shape=[
            jax.ShapeDtypeStruct((NPAD, H), jnp.float32),
            jax.ShapeDtypeStruct((NPAD,), jnp.float32),
        ],
        name="gcn_head",
    )(z0, deg0, deg1)


def _tc_mid_body(p0_ref, p1_ref, z_ref, dinv_ref, b_ref, w_ref, out_ref):
    dinv = dinv_ref[...]
    h = dinv[:, None] * (p0_ref[...] + p1_ref[...] + z_ref[...]) + b_ref[...]
    out_ref[...] = (h @ w_ref[...]) * dinv[:, None]


def _tc_mid(p, z, dinv, b, w):
    return pl.pallas_call(
        _tc_mid_body,
        grid=(NPAD // BLK,),
        in_specs=[
            pl.BlockSpec((BLK, H), lambda i: (i, 0)),
            pl.BlockSpec((BLK, H), lambda i: (i, 0)),
            pl.BlockSpec((BLK, H), lambda i: (i, 0)),
            pl.BlockSpec((BLK,), lambda i: (i,)),
            pl.BlockSpec((1, H), lambda i: (0, 0)),
            pl.BlockSpec((H, H), lambda i: (0, 0)),
        ],
        out_specs=pl.BlockSpec((BLK, H), lambda i: (i, 0)),
        out_shape=jax.ShapeDtypeStruct((NPAD, H), jnp.float32),
        name="gcn_mid",
    )(p[0], p[1], z, dinv, b, w)


def _tc_pre5_body(p0_ref, p1_ref, z_ref, dinv_ref, b_ref, out_ref):
    dinv = dinv_ref[...]
    h = dinv[:, None] * (p0_ref[...] + p1_ref[...] + z_ref[...]) + b_ref[...]
    out_ref[...] = h * dinv[:, None]


def _tc_pre5(p, z, dinv, b):
    return pl.pallas_call(
        _tc_pre5_body,
        grid=(NPAD // BLK,),
        in_specs=[
            pl.BlockSpec((BLK, H), lambda i: (i, 0)),
            pl.BlockSpec((BLK, H), lambda i: (i, 0)),
            pl.BlockSpec((BLK, H), lambda i: (i, 0)),
            pl.BlockSpec((BLK,), lambda i: (i,)),
            pl.BlockSpec((1, H), lambda i: (0, 0)),
        ],
        out_specs=pl.BlockSpec((BLK, H), lambda i: (i, 0)),
        out_shape=jax.ShapeDtypeStruct((NPAD, H), jnp.float32),
        name="gcn_pre5",
    )(p[0], p[1], z, dinv, b)


def _tc_tail_body(p0_ref, p1_ref, z_ref, dinv_ref, b_ref, w_ref, out_ref):
    dinv = dinv_ref[...]
    g = dinv[:, None] * (p0_ref[...] + p1_ref[...] + z_ref[...])
    out_ref[...] = g @ w_ref[...] + b_ref[...]


def _tc_tail(p, z, dinv, b5, w5):
    return pl.pallas_call(
        _tc_tail_body,
        grid=(NPAD // BLK,),
        in_specs=[
            pl.BlockSpec((BLK, H), lambda i: (i, 0)),
            pl.BlockSpec((BLK, H), lambda i: (i, 0)),
            pl.BlockSpec((BLK, H), lambda i: (i, 0)),
            pl.BlockSpec((BLK,), lambda i: (i,)),
            pl.BlockSpec((1, 128), lambda i: (0, 0)),
            pl.BlockSpec((H, 128), lambda i: (0, 0)),
        ],
        out_specs=pl.BlockSpec((BLK, 128), lambda i: (i, 0)),
        out_shape=jax.ShapeDtypeStruct((NPAD, 128), jnp.float32),
        name="gcn_tail",
    )(p[0], p[1], z, dinv, b5, w5)


@jax.jit
def kernel(x, edge_index, W1, b1, W2, b2, W3, b3, W4, b4, W5, b5):
    src = edge_index[0]
    dst = edge_index[1]
    pad = N_NODES + (jnp.arange(EPAD - N_EDGES, dtype=jnp.int32)
                     % (NPAD - N_NODES))
    src2d = jnp.concatenate([src, pad]).reshape(EPAD // 128, 128)
    dst2d = jnp.concatenate([dst, pad]).reshape(EPAD // 128, 128)
    xpad = jnp.pad(x, ((0, NPAD - N_NODES), (0, 0)))
    zeros2d = jnp.zeros((NPAD, H), jnp.float32)
    zeros1d = jnp.zeros((NPAD,), jnp.float32)

    z0 = _tc_mm1(xpad, W1)
    deg = _sc_degree(dst2d, zeros1d)
    z, dinv = _tc_head(z0, deg[0], deg[1])

    p = _sc_scatter(src2d, dst2d, z, zeros2d)
    z = _tc_mid(p, z, dinv, b1.reshape(1, H), W2)
    p = _sc_scatter(src2d, dst2d, z, zeros2d)
    z = _tc_mid(p, z, dinv, b2.reshape(1, H), W3)
    p = _sc_scatter(src2d, dst2d, z, zeros2d)
    z = _tc_mid(p, z, dinv, b3.reshape(1, H), W4)
    p = _sc_scatter(src2d, dst2d, z, zeros2d)
    z = _tc_pre5(p, z, dinv, b4.reshape(1, H))
    p = _sc_scatter(src2d, dst2d, z, zeros2d)
    out = _tc_tail(p, z, dinv, b5.reshape(1, 128), W5)
    return out[:N_NODES]


# speedup vs baseline: 1.0661x; 1.0287x over previous
import jax
import jax.numpy as jnp
from jax import lax
from jax.experimental import pallas as pl
from jax.experimental.pallas import tpu as pltpu
from jax.experimental.pallas import tpu_sc as plsc

N_NODES = 10000
N_EDGES = 320000
NPAD = 10240
EPAD = 327680
DUMMY = NPAD - 1
H = 32

NC, NS = 2, 16
NW = NC * NS
EROWS_PER_TILE = EPAD // NW // 128
CH = 5
NROWS_PER_TILE = NPAD // NS


NCHUNK = EROWS_PER_TILE // CH
NBUF = 3


def _sc_scatter_body(src_hbm, dst_hbm, z_hbm, zero_hbm, out_hbm,
                     idx_s, idx_d, rows, accum, gsem, ssem):
    c = lax.axis_index("c")
    s = lax.axis_index("s")
    tslice = pl.ds(s * NROWS_PER_TILE, NROWS_PER_TILE)

    zcp = pltpu.async_copy(zero_hbm.at[tslice], accum.at[tslice], gsem)
    row0 = (c * NS + s) * EROWS_PER_TILE
    pltpu.sync_copy(src_hbm.at[pl.ds(row0, EROWS_PER_TILE)], idx_s)
    pltpu.sync_copy(dst_hbm.at[pl.ds(row0, EROWS_PER_TILE)], idx_d)
    zcp.wait()
    plsc.subcore_barrier()

    gathers = {}
    adds = {}

    def fire(ci):
        buf = rows.at[ci % NBUF]
        gathers[ci] = [
            pltpu.async_copy(z_hbm.at[idx_s.at[ci * CH + j]], buf.at[j], gsem)
            for j in range(CH)]

    for ci in range(NBUF - 1):
        fire(ci)
    for ci in range(NCHUNK):
        buf = rows.at[ci % NBUF]
        for cp in gathers.pop(ci):
            cp.wait()
        if ci >= 1:
            for cp in adds.pop(ci - 1):
                cp.wait()
        adds[ci] = [
            pltpu.async_copy(buf.at[j], accum.at[idx_d.at[ci * CH + j]],
                             ssem, add=True)
            for j in range(CH)]
        if ci + NBUF - 1 < NCHUNK:
            fire(ci + NBUF - 1)
    for cp in adds.pop(NCHUNK - 1):
        cp.wait()

    plsc.subcore_barrier()
    pltpu.sync_copy(accum.at[tslice], out_hbm.at[c].at[tslice])


_sc_scatter = pl.kernel(
    _sc_scatter_body,
    out_type=jax.ShapeDtypeStruct((NC, NPAD, H), jnp.float32),
    mesh=plsc.VectorSubcoreMesh(core_axis_name="c", subcore_axis_name="s"),
    scratch_types=[
        pltpu.VMEM((EROWS_PER_TILE, 128), jnp.int32),
        pltpu.VMEM((EROWS_PER_TILE, 128), jnp.int32),
        pltpu.VMEM((NBUF, CH, 128, H), jnp.float32),
        pltpu.VMEM_SHARED((NPAD, H), jnp.float32),
        pltpu.SemaphoreType.DMA,
        pltpu.SemaphoreType.DMA,
    ],
    compiler_params=pltpu.CompilerParams(use_tc_tiling_on_sc=False),
    name="gcn_edge_scatter",
)


def _sc_degree_body(dst_hbm, zero_hbm, out_hbm, idx_d, ones, accum, sem):
    c = lax.axis_index("c")
    s = lax.axis_index("s")
    tslice = pl.ds(s * NROWS_PER_TILE, NROWS_PER_TILE)

    one16 = jnp.ones((16,), jnp.float32)
    def oloop(i, _):
        ones[pl.ds(i * 16, 16)] = one16
        return ()
    lax.fori_loop(0, 128 // 16, oloop, ())
    row0 = (c * NS + s) * EROWS_PER_TILE
    pltpu.sync_copy(dst_hbm.at[pl.ds(row0, EROWS_PER_TILE)], idx_d)
    pltpu.sync_copy(zero_hbm.at[tslice], accum.at[tslice])
    plsc.subcore_barrier()

    adds = [pltpu.async_copy(ones, accum.at[idx_d.at[r]], sem, add=True)
            for r in range(EROWS_PER_TILE)]
    for cp in adds:
        cp.wait()

    plsc.subcore_barrier()
    pltpu.sync_copy(accum.at[tslice], out_hbm.at[c].at[tslice])


_sc_degree = pl.kernel(
    _sc_degree_body,
    out_type=jax.ShapeDtypeStruct((NC, NPAD), jnp.float32),
    mesh=plsc.VectorSubcoreMesh(core_axis_name="c", subcore_axis_name="s"),
    scratch_types=[
        pltpu.VMEM((EROWS_PER_TILE, 128), jnp.int32),
        pltpu.VMEM((128,), jnp.float32),
        pltpu.VMEM_SHARED((NPAD,), jnp.float32),
        pltpu.SemaphoreType.DMA,
    ],
    compiler_params=pltpu.CompilerParams(use_tc_tiling_on_sc=False),
    name="gcn_degree",
)


BLK = 1024


def _tc_mm1_body(x_ref, w1_ref, z_ref):
    z_ref[...] = x_ref[...] @ w1_ref[...]


def _tc_mm1(x, w1):
    return pl.pallas_call(
        _tc_mm1_body,
        grid=(NPAD // BLK,),
        in_specs=[
            pl.BlockSpec((BLK, 128), lambda i: (i, 0)),
            pl.BlockSpec((128, H), lambda i: (0, 0)),
        ],
        out_specs=pl.BlockSpec((BLK, H), lambda i: (i, 0)),
        out_shape=jax.ShapeDtypeStruct((NPAD, H), jnp.float32),
        name="gcn_mm1",
    )(x, w1)


def _tc_head_body(z0_ref, d0_ref, d1_ref, z_ref, dinv_ref):
    deg = d0_ref[...] + d1_ref[...] + 1.0
    dinv = lax.rsqrt(deg)
    dinv_ref[...] = dinv
    z_ref[...] = z0_ref[...] * dinv[:, None]


def _tc_head(z0, deg0, deg1):
    return pl.pallas_call(
        _tc_head_body,
        grid=(NPAD // BLK,),
        in_specs=[
            pl.BlockSpec((BLK, H), lambda i: (i, 0)),
            pl.BlockSpec((BLK,), lambda i: (i,)),
            pl.BlockSpec((BLK,), lambda i: (i,)),
        ],
        out_specs=[
            pl.BlockSpec((BLK, H), lambda i: (i, 0)),
            pl.BlockSpec((BLK,), lambda i: (i,)),
        ],
        out_shape=[
            jax.ShapeDtypeStruct((NPAD, H), jnp.float32),
            jax.ShapeDtypeStruct((NPAD,), jnp.float32),
        ],
        name="gcn_head",
    )(z0, deg0, deg1)


def _tc_mid_body(p0_ref, p1_ref, z_ref, dinv_ref, b_ref, w_ref, out_ref):
    dinv = dinv_ref[...]
    h = dinv[:, None] * (p0_ref[...] + p1_ref[...] + z_ref[...]) + b_ref[...]
    out_ref[...] = (h @ w_ref[...]) * dinv[:, None]


def _tc_mid(p, z, dinv, b, w):
    return pl.pallas_call(
        _tc_mid_body,
        grid=(NPAD // BLK,),
        in_specs=[
            pl.BlockSpec((BLK, H), lambda i: (i, 0)),
            pl.BlockSpec((BLK, H), lambda i: (i, 0)),
            pl.BlockSpec((BLK, H), lambda i: (i, 0)),
            pl.BlockSpec((BLK,), lambda i: (i,)),
            pl.BlockSpec((1, H), lambda i: (0, 0)),
            pl.BlockSpec((H, H), lambda i: (0, 0)),
        ],
        out_specs=pl.BlockSpec((BLK, H), lambda i: (i, 0)),
        out_shape=jax.ShapeDtypeStruct((NPAD, H), jnp.float32),
        name="gcn_mid",
    )(p[0], p[1], z, dinv, b, w)


def _tc_pre5_body(p0_ref, p1_ref, z_ref, dinv_ref, b_ref, out_ref):
    dinv = dinv_ref[...]
    h = dinv[:, None] * (p0_ref[...] + p1_ref[...] + z_ref[...]) + b_ref[...]
    out_ref[...] = h * dinv[:, None]


def _tc_pre5(p, z, dinv, b):
    return pl.pallas_call(
        _tc_pre5_body,
        grid=(NPAD // BLK,),
        in_specs=[
            pl.BlockSpec((BLK, H), lambda i: (i, 0)),
            pl.BlockSpec((BLK, H), lambda i: (i, 0)),
            pl.BlockSpec((BLK, H), lambda i: (i, 0)),
            pl.BlockSpec((BLK,), lambda i: (i,)),
            pl.BlockSpec((1, H), lambda i: (0, 0)),
        ],
        out_specs=pl.BlockSpec((BLK, H), lambda i: (i, 0)),
        out_shape=jax.ShapeDtypeStruct((NPAD, H), jnp.float32),
        name="gcn_pre5",
    )(p[0], p[1], z, dinv, b)


def _tc_tail_body(p0_ref, p1_ref, z_ref, dinv_ref, b_ref, w_ref, out_ref):
    dinv = dinv_ref[...]
    g = dinv[:, None] * (p0_ref[...] + p1_ref[...] + z_ref[...])
    out_ref[...] = g @ w_ref[...] + b_ref[...]


def _tc_tail(p, z, dinv, b5, w5):
    return pl.pallas_call(
        _tc_tail_body,
        grid=(NPAD // BLK,),
        in_specs=[
            pl.BlockSpec((BLK, H), lambda i: (i, 0)),
            pl.BlockSpec((BLK, H), lambda i: (i, 0)),
            pl.BlockSpec((BLK, H), lambda i: (i, 0)),
            pl.BlockSpec((BLK,), lambda i: (i,)),
            pl.BlockSpec((1, 128), lambda i: (0, 0)),
            pl.BlockSpec((H, 128), lambda i: (0, 0)),
        ],
        out_specs=pl.BlockSpec((BLK, 128), lambda i: (i, 0)),
        out_shape=jax.ShapeDtypeStruct((NPAD, 128), jnp.float32),
        name="gcn_tail",
    )(p[0], p[1], z, dinv, b5, w5)


@jax.jit
def kernel(x, edge_index, W1, b1, W2, b2, W3, b3, W4, b4, W5, b5):
    src = edge_index[0]
    dst = edge_index[1]
    pad = N_NODES + (jnp.arange(EPAD - N_EDGES, dtype=jnp.int32)
                     % (NPAD - N_NODES))
    src2d = jnp.concatenate([src, pad]).reshape(EPAD // 128, 128)
    dst2d = jnp.concatenate([dst, pad]).reshape(EPAD // 128, 128)
    xpad = jnp.pad(x, ((0, NPAD - N_NODES), (0, 0)))
    zeros2d = jnp.zeros((NPAD, H), jnp.float32)
    zeros1d = jnp.zeros((NPAD,), jnp.float32)

    z0 = _tc_mm1(xpad, W1)
    deg = _sc_degree(dst2d, zeros1d)
    z, dinv = _tc_head(z0, deg[0], deg[1])

    p = _sc_scatter(src2d, dst2d, z, zeros2d)
    z = _tc_mid(p, z, dinv, b1.reshape(1, H), W2)
    p = _sc_scatter(src2d, dst2d, z, zeros2d)
    z = _tc_mid(p, z, dinv, b2.reshape(1, H), W3)
    p = _sc_scatter(src2d, dst2d, z, zeros2d)
    z = _tc_mid(p, z, dinv, b3.reshape(1, H), W4)
    p = _sc_scatter(src2d, dst2d, z, zeros2d)
    z = _tc_pre5(p, z, dinv, b4.reshape(1, H))
    p = _sc_scatter(src2d, dst2d, z, zeros2d)
    out = _tc_tail(p, z, dinv, b5.reshape(1, 128), W5)
    return out[:N_NODES]
